# baseline (speedup 1.0000x reference)
import functools

import jax
import jax.numpy as jnp
from jax import lax
from jax.experimental import pallas as pl
from jax.experimental.pallas import tpu as pltpu
from jax.experimental.pallas import tpu_sc as plsc

N = 8192
D = 64
K = 8192
TILE_M = 2048
TILE_K = 1024
LANES = 128
NB = TILE_K // LANES
M_TILES = N // TILE_M
K_CHUNKS = K // TILE_K


def _argmin_body(xt_ref, wt_ref, b_ref, cbt_ref, cn_ref, idx_ref,
                 qn_ref, ssb_ref, rmin_ref, rblk_ref):
    kc = pl.program_id(1)

    @pl.when(kc == 0)
    def _init():
        q = jnp.dot(xt_ref[...], wt_ref[...]) + b_ref[...]
        ss0 = jnp.sum(q * q, axis=1, keepdims=True)
        qn = q / jnp.maximum(jnp.sqrt(ss0), 1e-12)
        qn_ref[...] = qn
        ssn = jnp.sum(qn * qn, axis=1, keepdims=True)
        ssb_ref[...] = jnp.broadcast_to(ssn, (TILE_M, LANES))

    dot2 = jnp.dot(qn_ref[...], cbt_ref[...])
    ssb = ssb_ref[...]

    @pl.when(kc == 0)
    def _first():
        for j in range(NB):
            sl = slice(j * LANES, (j + 1) * LANES)
            t1 = ssb + cn_ref[:, sl]
            rmin_ref[:, sl] = t1 + dot2[:, sl]
        rblk_ref[...] = jnp.broadcast_to(
            lax.broadcasted_iota(jnp.int32, (1, TILE_K), 1) // LANES,
            (TILE_M, TILE_K))

    @pl.when(kc > 0)
    def _update():
        for j in range(NB):
            sl = slice(j * LANES, (j + 1) * LANES)
            t1 = ssb + cn_ref[:, sl]
            s = t1 + dot2[:, sl]
            m = s < rmin_ref[:, sl]
            rmin_ref[:, sl] = jnp.where(m, s, rmin_ref[:, sl])
            rblk_ref[:, sl] = jnp.where(m, kc * NB + j, rblk_ref[:, sl])

    @pl.when(kc == K_CHUNKS - 1)
    def _flush():
        lane = lax.broadcasted_iota(jnp.int32, (TILE_M, LANES), 1)
        pairs = []
        for j in range(NB):
            sl = slice(j * LANES, (j + 1) * LANES)
            g = rblk_ref[:, sl] * LANES + lane
            pairs.append((rmin_ref[:, sl], g))

        def comb(left, right):
            lv, lg = left
            rv, rg = right
            m = jnp.logical_or(
                rv < lv, jnp.logical_and(rv == lv, rg < lg))
            return (jnp.where(m, rv, lv), jnp.where(m, rg, lg))

        while len(pairs) > 1:
            pairs = [comb(pairs[i], pairs[i + 1])
                     for i in range(0, len(pairs), 2)]
        fv, fg = pairs[0]
        mv = jnp.min(fv, axis=1, keepdims=True)
        gm = jnp.where(fv == mv, fg, K)
        idx_ref[0, 0, :] = jnp.min(gm, axis=1)


def _compute_indices(xflat, conv_wt, conv_b2, cbt_m2, cn2):
    return pl.pallas_call(
        _argmin_body,
        grid=(M_TILES, K_CHUNKS),
        in_specs=[
            pl.BlockSpec((TILE_M, D), lambda i, k: (i, 0)),
            pl.BlockSpec((D, D), lambda i, k: (0, 0)),
            pl.BlockSpec((1, D), lambda i, k: (0, 0)),
            pl.BlockSpec((D, TILE_K), lambda i, k: (0, k)),
            pl.BlockSpec((1, TILE_K), lambda i, k: (0, k)),
        ],
        out_specs=pl.BlockSpec((1, 1, TILE_M), lambda i, k: (i, 0, 0)),
        out_shape=jax.ShapeDtypeStruct((M_TILES, 1, TILE_M), jnp.int32),
        scratch_shapes=[
            pltpu.VMEM((TILE_M, D), jnp.float32),
            pltpu.VMEM((TILE_M, LANES), jnp.float32),
            pltpu.VMEM((TILE_M, TILE_K), jnp.float32),
            pltpu.VMEM((TILE_M, TILE_K), jnp.int32),
        ],
    )(xflat, conv_wt, conv_b2, cbt_m2, cn2)


def _sc_gather(table128, idx):
    info = plsc.get_sparse_core_info()
    nw = info.num_cores * info.num_subcores
    b_per_w = N // nw
    G = 8
    rows_g = b_per_w // G
    mesh = plsc.VectorSubcoreMesh(core_axis_name="c", subcore_axis_name="s")

    @functools.partial(
        pl.kernel, mesh=mesh,
        out_type=jax.ShapeDtypeStruct((N, 128), jnp.float32),
        scratch_types=[
            pltpu.VMEM((b_per_w,), jnp.int32),
            pltpu.VMEM((b_per_w, 128), jnp.float32),
            pltpu.VMEM_SHARED((K, 128), jnp.float32),
            pltpu.SemaphoreType.DMA,
        ],
    )
    def k(table_hbm, idx_hbm, out_hbm, idx_v, rows_v, table_s, sem):
        wid = lax.axis_index("s") * info.num_cores + lax.axis_index("c")
        base = wid * b_per_w
        sid = lax.axis_index("s")
        stripe = K // info.num_subcores
        pltpu.sync_copy(table_hbm.at[pl.ds(sid * stripe, stripe)],
                        table_s.at[pl.ds(sid * stripe, stripe)])
        pltpu.sync_copy(idx_hbm.at[pl.ds(base, b_per_w)], idx_v)
        plsc.subcore_barrier()
        copies = []
        for g in range(G):
            copies.append(pltpu.async_copy(
                table_s.at[idx_v.at[pl.ds(g * rows_g, rows_g)]],
                rows_v.at[pl.ds(g * rows_g, rows_g)], sem))
        for c in copies:
            c.wait()
        pltpu.sync_copy(rows_v, out_hbm.at[pl.ds(base, b_per_w)])

    return k(table128, idx)


def kernel(input, conv_w, conv_b, codebook):
    B, C, H, W = input.shape
    xflat = jnp.transpose(input, (0, 2, 3, 1)).reshape(N, D)
    cbt_m2 = (-2.0 * codebook).T
    cn2 = jnp.sum(codebook * codebook, axis=1).reshape(1, K)
    idx3 = _compute_indices(xflat, conv_w.T, conv_b.reshape(1, D),
                            cbt_m2, cn2)
    idx = idx3.reshape(N)
    table128 = jnp.pad(codebook, ((0, 0), (0, 128 - D)))
    rows = _sc_gather(table128, idx)[:, :D]
    return jnp.transpose(rows.reshape(B, H, W, D), (0, 3, 1, 2))


# speedup vs baseline: 2.5609x; 2.5609x over previous
import functools

import jax
import jax.numpy as jnp
from jax import lax
from jax.experimental import pallas as pl
from jax.experimental.pallas import tpu as pltpu
from jax.experimental.pallas import tpu_sc as plsc

N = 8192
D = 64
K = 8192
TILE_M = 8192
TILE_K = 1024
LANES = 128
NB = TILE_K // LANES
M_TILES = N // TILE_M
K_CHUNKS = K // TILE_K


def _argmin_body(xt_ref, wt_ref, b_ref, cbt_ref, cn_ref, idx_ref,
                 qn_ref, ssb_ref, rmin_ref, rblk_ref):
    kc = pl.program_id(1)

    @pl.when(kc == 0)
    def _init():
        q = jnp.dot(xt_ref[...], wt_ref[...]) + b_ref[...]
        ss0 = jnp.sum(q * q, axis=1, keepdims=True)
        qn = q / jnp.maximum(jnp.sqrt(ss0), 1e-12)
        qn_ref[...] = qn
        ssn = jnp.sum(qn * qn, axis=1, keepdims=True)
        ssb_ref[...] = jnp.broadcast_to(ssn, (TILE_M, LANES))
        rmin_ref[...] = jnp.full((TILE_M, LANES), jnp.inf, jnp.float32)
        rblk_ref[...] = jnp.zeros((TILE_M, LANES), jnp.int32)

    dot2 = jnp.dot(qn_ref[...], cbt_ref[...])
    ssb = ssb_ref[...]
    pairs = []
    for j in range(NB):
        t1 = ssb + cn_ref[:, j * LANES:(j + 1) * LANES]
        s = t1 + dot2[:, j * LANES:(j + 1) * LANES]
        pairs.append((s, kc * NB + j))

    def comb(left, right):
        lv, lb = left
        rv, rb = right
        m = rv < lv
        return (jnp.where(m, rv, lv), jnp.where(m, rb, lb))

    while len(pairs) > 1:
        pairs = [comb(pairs[i], pairs[i + 1])
                 for i in range(0, len(pairs), 2)]
    sv, sb = pairs[0]
    m = sv < rmin_ref[...]
    rmin = jnp.where(m, sv, rmin_ref[...])
    rblk = jnp.where(m, sb, rblk_ref[...])
    rmin_ref[...] = rmin
    rblk_ref[...] = rblk

    @pl.when(kc == K_CHUNKS - 1)
    def _flush():
        lane = lax.broadcasted_iota(jnp.int32, (TILE_M, LANES), 1)
        g = rblk * LANES + lane
        mv = jnp.min(rmin, axis=1, keepdims=True)
        gm = jnp.where(rmin == mv, g, K)
        idx_ref[0, 0, :] = jnp.min(gm, axis=1)


def _compute_indices(xflat, conv_wt, conv_b2, cbt_m2, cn2):
    return pl.pallas_call(
        _argmin_body,
        grid=(M_TILES, K_CHUNKS),
        in_specs=[
            pl.BlockSpec((TILE_M, D), lambda i, k: (i, 0)),
            pl.BlockSpec((D, D), lambda i, k: (0, 0)),
            pl.BlockSpec((1, D), lambda i, k: (0, 0)),
            pl.BlockSpec((D, TILE_K), lambda i, k: (0, k)),
            pl.BlockSpec((1, TILE_K), lambda i, k: (0, k)),
        ],
        out_specs=pl.BlockSpec((1, 1, TILE_M), lambda i, k: (i, 0, 0)),
        out_shape=jax.ShapeDtypeStruct((M_TILES, 1, TILE_M), jnp.int32),
        scratch_shapes=[
            pltpu.VMEM((TILE_M, D), jnp.float32),
            pltpu.VMEM((TILE_M, LANES), jnp.float32),
            pltpu.VMEM((TILE_M, LANES), jnp.float32),
            pltpu.VMEM((TILE_M, LANES), jnp.int32),
        ],
    )(xflat, conv_wt, conv_b2, cbt_m2, cn2)


def _sc_gather(table128, idx):
    info = plsc.get_sparse_core_info()
    nw = info.num_cores * info.num_subcores
    b_per_w = N // nw
    G = 8
    rows_g = b_per_w // G
    mesh = plsc.VectorSubcoreMesh(core_axis_name="c", subcore_axis_name="s")

    @functools.partial(
        pl.kernel, mesh=mesh,
        out_type=jax.ShapeDtypeStruct((N, 128), jnp.float32),
        scratch_types=[
            pltpu.VMEM((b_per_w,), jnp.int32),
            pltpu.VMEM((b_per_w, 128), jnp.float32),
            pltpu.VMEM_SHARED((K, 128), jnp.float32),
            pltpu.SemaphoreType.DMA,
        ],
    )
    def k(table_hbm, idx_hbm, out_hbm, idx_v, rows_v, table_s, sem):
        wid = lax.axis_index("s") * info.num_cores + lax.axis_index("c")
        base = wid * b_per_w
        sid = lax.axis_index("s")
        stripe = K // info.num_subcores
        pltpu.sync_copy(table_hbm.at[pl.ds(sid * stripe, stripe)],
                        table_s.at[pl.ds(sid * stripe, stripe)])
        pltpu.sync_copy(idx_hbm.at[pl.ds(base, b_per_w)], idx_v)
        plsc.subcore_barrier()
        copies = []
        for g in range(G):
            copies.append(pltpu.async_copy(
                table_s.at[idx_v.at[pl.ds(g * rows_g, rows_g)]],
                rows_v.at[pl.ds(g * rows_g, rows_g)], sem))
        for c in copies:
            c.wait()
        pltpu.sync_copy(rows_v, out_hbm.at[pl.ds(base, b_per_w)])

    return k(table128, idx)


def kernel(input, conv_w, conv_b, codebook):
    B, C, H, W = input.shape
    xflat = jnp.transpose(input, (0, 2, 3, 1)).reshape(N, D)
    cbt_m2 = (-2.0 * codebook).T
    cn2 = jnp.sum(codebook * codebook, axis=1).reshape(1, K)
    idx3 = _compute_indices(xflat, conv_w.T, conv_b.reshape(1, D),
                            cbt_m2, cn2)
    idx = idx3.reshape(N)
    table128 = jnp.pad(codebook, ((0, 0), (0, 128 - D)))
    rows = _sc_gather(table128, idx)[:, :D]
    return jnp.transpose(rows.reshape(B, H, W, D), (0, 3, 1, 2))
